# Initial kernel scaffold; baseline (speedup 1.0000x reference)
import functools

import jax
import jax.numpy as jnp
from jax import lax
from jax.experimental import pallas as pl
from jax.experimental.pallas import tpu as pltpu
from jax.experimental.pallas import tpu_sc as plsc

N_NODES = 10000
N_EDGES = 160000
NC = 2
NS = 16
LANES = 16
CHUNK = 80
EDGES_PER_TILE = N_EDGES // NS
NCHUNKS = EDGES_PER_TILE // CHUNK
ROWS_PER_TILE = N_NODES // NS
INIT_ROWS = 125


def _matmul_x(x, w, bm=1000):
    n, k = x.shape
    d = w.shape[1]

    def body(x_ref, w_ref, o_ref):
        o_ref[...] = jnp.dot(x_ref[...], w_ref[...],
                             preferred_element_type=jnp.float32)

    return pl.pallas_call(
        body,
        grid=(n // bm,),
        in_specs=[
            pl.BlockSpec((bm, k), lambda i: (i, 0)),
            pl.BlockSpec((k, d), lambda i: (0, 0)),
        ],
        out_specs=pl.BlockSpec((bm, d), lambda i: (i, 0)),
        out_shape=jax.ShapeDtypeStruct((n, d), jnp.float32),
    )(x, w)


def _matmul_relu(a2, b2, w, bm=1000):
    _, n, h = a2.shape
    d = w.shape[1]

    def body(a_ref, b_ref, w_ref, o_ref):
        h0 = jnp.maximum(a_ref[0] + b_ref[0], 0.0)
        h1 = jnp.maximum(a_ref[1] + b_ref[1], 0.0)
        o_ref[...] = (
            jnp.dot(h0, w_ref[:h, :], preferred_element_type=jnp.float32)
            + jnp.dot(h1, w_ref[h:, :], preferred_element_type=jnp.float32)
        )

    return pl.pallas_call(
        body,
        grid=(n // bm,),
        in_specs=[
            pl.BlockSpec((2, bm, h), lambda i: (0, i, 0)),
            pl.BlockSpec((2, h), lambda i: (0, 0)),
            pl.BlockSpec((2 * h, d), lambda i: (0, 0)),
        ],
        out_specs=pl.BlockSpec((bm, d), lambda i: (i, 0)),
        out_shape=jax.ShapeDtypeStruct((n, d), jnp.float32),
    )(a2, b2, w)


def _spmm_sc(sup2x, src2, dst, ew, bias2, dh):
    dblks = dh // LANES
    mesh = plsc.VectorSubcoreMesh(core_axis_name="c", subcore_axis_name="s",
                                  num_cores=NC, num_subcores=NS)

    @functools.partial(
        pl.kernel,
        out_type=jax.ShapeDtypeStruct((NC, N_NODES, dh), jnp.float32),
        mesh=mesh,
        scratch_types=[
            pltpu.VMEM((CHUNK, dh), jnp.float32),
            pltpu.VMEM((CHUNK,), jnp.int32),
            pltpu.VMEM((CHUNK,), jnp.int32),
            pltpu.VMEM((CHUNK,), jnp.float32),
            pltpu.VMEM((dh,), jnp.float32),
            pltpu.VMEM((INIT_ROWS, dh), jnp.float32),
            pltpu.VMEM_SHARED((N_NODES, dh), jnp.float32),
            pltpu.SemaphoreType.DMA,
        ],
    )
    def k(sup_hbm, src_hbm, dst_hbm, ew_hbm, b_hbm, out_hbm,
          rows_v, si_v, di_v, w_v, bvec_v, ib_v, acc_sh, sem):
        c = lax.axis_index("c")
        s = lax.axis_index("s")

        pltpu.sync_copy(b_hbm.at[c], bvec_v)

        def init_row(r, carry):
            for d in range(dblks):
                sl = pl.ds(d * LANES, LANES)
                ib_v[r, sl] = bvec_v[sl]
            return carry

        lax.fori_loop(0, INIT_ROWS, init_row, 0)
        row0 = s * ROWS_PER_TILE
        for j in range(ROWS_PER_TILE // INIT_ROWS):
            pltpu.sync_copy(ib_v, acc_sh.at[pl.ds(row0 + j * INIT_ROWS,
                                                  INIT_ROWS)])
        plsc.subcore_barrier()

        def chunk_body(kk, carry):
            base = s * EDGES_PER_TILE + kk * CHUNK
            pltpu.sync_copy(src_hbm.at[c, pl.ds(base, CHUNK)], si_v)
            pltpu.sync_copy(dst_hbm.at[pl.ds(base, CHUNK)], di_v)
            pltpu.sync_copy(ew_hbm.at[pl.ds(base, CHUNK)], w_v)
            pltpu.async_copy(sup_hbm.at[si_v], rows_v, sem).wait()

            def scale_edge(e, cc):
                wv = w_v[e]
                for d in range(dblks):
                    sl = pl.ds(d * LANES, LANES)
                    rows_v[e, sl] = rows_v[e, sl] * wv
                return cc

            lax.fori_loop(0, CHUNK, scale_edge, 0)
            pltpu.sync_copy(rows_v, acc_sh.at[di_v], add=True)
            return carry

        lax.fori_loop(0, NCHUNKS, chunk_body, 0)
        plsc.subcore_barrier()

        pltpu.sync_copy(acc_sh.at[pl.ds(row0, ROWS_PER_TILE)],
                        out_hbm.at[c, pl.ds(row0, ROWS_PER_TILE)])

    return k(sup2x, src2, dst, ew, bias2)


def kernel(x, edge_index, edge_weight, W1, b1, W2, b2, W3, b3):
    ei = edge_index.astype(jnp.int32)
    src2 = jnp.stack([ei[0] * 2, ei[0] * 2 + 1])
    dst = ei[1]
    zeros128 = jnp.zeros((2, 128), jnp.float32)

    s1 = _matmul_x(x, W1)
    a1 = _spmm_sc(s1.reshape(2 * N_NODES, 128), src2, dst,
                  edge_weight, zeros128, 128)
    s2 = _matmul_relu(a1, b1.reshape(2, 128), W2)
    a2 = _spmm_sc(s2.reshape(2 * N_NODES, 128), src2, dst,
                  edge_weight, zeros128, 128)
    s3 = _matmul_relu(a2, b2.reshape(2, 128), W3)
    a3 = _spmm_sc(s3.reshape(2 * N_NODES, 64), src2, dst,
                  edge_weight, b3.reshape(2, 64), 64)
    return jnp.concatenate([a3[0], a3[1]], axis=1)


# speedup vs baseline: 1.9286x; 1.9286x over previous
import functools

import jax
import jax.numpy as jnp
from jax import lax
from jax.experimental import pallas as pl
from jax.experimental.pallas import tpu as pltpu
from jax.experimental.pallas import tpu_sc as plsc

N_NODES = 10000
N_EDGES = 160000
EPAD = 163840
NC = 2
NS = 16
LANES = 16
INIT_ROWS = 125


def _matmul_x(x, w, bm=1000):
    n, k = x.shape
    d = w.shape[1]

    def body(x_ref, w_ref, o_ref):
        o_ref[...] = jnp.dot(x_ref[...], w_ref[...],
                             preferred_element_type=jnp.float32)

    return pl.pallas_call(
        body,
        grid=(n // bm,),
        in_specs=[
            pl.BlockSpec((bm, k), lambda i: (i, 0)),
            pl.BlockSpec((k, d), lambda i: (0, 0)),
        ],
        out_specs=pl.BlockSpec((bm, d), lambda i: (i, 0)),
        out_shape=jax.ShapeDtypeStruct((n, d), jnp.float32),
    )(x, w)


def _matmul_relu(a2, b2, w, bm=1000):
    _, n, h = a2.shape
    d = w.shape[1]

    def body(a_ref, b_ref, w_ref, o_ref):
        h0 = jnp.maximum(a_ref[0] + b_ref[0], 0.0)
        h1 = jnp.maximum(a_ref[1] + b_ref[1], 0.0)
        o_ref[...] = (
            jnp.dot(h0, w_ref[:h, :], preferred_element_type=jnp.float32)
            + jnp.dot(h1, w_ref[h:, :], preferred_element_type=jnp.float32)
        )

    return pl.pallas_call(
        body,
        grid=(n // bm,),
        in_specs=[
            pl.BlockSpec((2, bm, h), lambda i: (0, i, 0)),
            pl.BlockSpec((2, h), lambda i: (0, 0)),
            pl.BlockSpec((2 * h, d), lambda i: (0, 0)),
        ],
        out_specs=pl.BlockSpec((bm, d), lambda i: (i, 0)),
        out_shape=jax.ShapeDtypeStruct((n, d), jnp.float32),
    )(a2, b2, w)


def _combine_bias(p2, b, bm=1000):
    _, n, d = p2.shape

    def body(p_ref, b_ref, o_ref):
        o_ref[...] = p_ref[0] + p_ref[1] + b_ref[...]

    return pl.pallas_call(
        body,
        grid=(n // bm,),
        in_specs=[
            pl.BlockSpec((2, bm, d), lambda i: (0, i, 0)),
            pl.BlockSpec((1, d), lambda i: (0, 0)),
        ],
        out_specs=pl.BlockSpec((bm, d), lambda i: (i, 0)),
        out_shape=jax.ShapeDtypeStruct((n, d), jnp.float32),
    )(p2, b)


def _spmm_sc(sup, srcflat, dst, ew, feat_split):
    dh = sup.shape[1]
    dblks = dh // LANES
    chunk = 80
    if feat_split:
        per_tile = EPAD // NS
    else:
        per_tile = EPAD // (NC * NS)
    nchunks = per_tile // chunk
    rows_per_tile = N_NODES // NS
    mesh = plsc.VectorSubcoreMesh(core_axis_name="c", subcore_axis_name="s",
                                  num_cores=NC, num_subcores=NS)

    @functools.partial(
        pl.kernel,
        out_type=jax.ShapeDtypeStruct((NC, N_NODES, dh), jnp.float32),
        mesh=mesh,
        scratch_types=[
            pltpu.VMEM((chunk, dh), jnp.float32),
            pltpu.VMEM((chunk,), jnp.int32),
            pltpu.VMEM((chunk,), jnp.int32),
            pltpu.VMEM((chunk,), jnp.float32),
            pltpu.VMEM((INIT_ROWS, dh), jnp.float32),
            pltpu.VMEM_SHARED((N_NODES, dh), jnp.float32),
            pltpu.SemaphoreType.DMA,
        ],
    )
    def k(sup_hbm, src_hbm, dst_hbm, ew_hbm, out_hbm,
          rows_v, si_v, di_v, w_v, zb_v, acc_sh, sem):
        c = lax.axis_index("c")
        s = lax.axis_index("s")

        def init_row(r, carry):
            for d in range(dblks):
                zb_v[r, pl.ds(d * LANES, LANES)] = jnp.zeros(
                    (LANES,), jnp.float32)
            return carry

        lax.fori_loop(0, INIT_ROWS, init_row, 0)
        row0 = s * rows_per_tile
        for j in range(rows_per_tile // INIT_ROWS):
            pltpu.sync_copy(zb_v, acc_sh.at[pl.ds(row0 + j * INIT_ROWS,
                                                  INIT_ROWS)])
        plsc.subcore_barrier()

        if feat_split:
            ebase0 = s * per_tile
            sbase0 = c * EPAD + ebase0
        else:
            ebase0 = (c * NS + s) * per_tile
            sbase0 = ebase0

        def chunk_body(kk, carry):
            off = kk * chunk
            pltpu.sync_copy(src_hbm.at[pl.ds(sbase0 + off, chunk)], si_v)
            pltpu.sync_copy(dst_hbm.at[pl.ds(ebase0 + off, chunk)], di_v)
            pltpu.sync_copy(ew_hbm.at[pl.ds(ebase0 + off, chunk)], w_v)
            pltpu.async_copy(sup_hbm.at[si_v], rows_v, sem).wait()

            def scale_group(g, cc):
                wvec = w_v[pl.ds(g * LANES, LANES)]
                for j in range(LANES):
                    e = g * LANES + j
                    wj = wvec[j]
                    for d in range(dblks):
                        sl = pl.ds(d * LANES, LANES)
                        rows_v[e, sl] = rows_v[e, sl] * wj
                return cc

            lax.fori_loop(0, chunk // LANES, scale_group, 0)
            pltpu.sync_copy(rows_v, acc_sh.at[di_v], add=True)
            return carry

        lax.fori_loop(0, nchunks, chunk_body, 0)
        plsc.subcore_barrier()

        out_rows = rows_per_tile // 8 * 8
        pltpu.sync_copy(acc_sh.at[pl.ds(s * out_rows, out_rows)],
                        out_hbm.at[c, pl.ds(s * out_rows, out_rows)])
        tail = N_NODES - NS * out_rows

        @pl.when(s == 0)
        def _():
            pltpu.sync_copy(acc_sh.at[pl.ds(NS * out_rows, tail)],
                            out_hbm.at[c, pl.ds(NS * out_rows, tail)])

    return k(sup, srcflat, dst, ew)


def kernel(x, edge_index, edge_weight, W1, b1, W2, b2, W3, b3):
    ei = edge_index.astype(jnp.int32)
    pad = EPAD - N_EDGES
    src = jnp.concatenate([ei[0], jnp.zeros((pad,), jnp.int32)])
    dst = jnp.concatenate([ei[1], jnp.zeros((pad,), jnp.int32)])
    ew = jnp.concatenate([edge_weight, jnp.zeros((pad,), jnp.float32)])
    src2 = jnp.concatenate([src * 2, src * 2 + 1])

    s1 = _matmul_x(x, W1)
    a1 = _spmm_sc(s1.reshape(2 * N_NODES, 128), src2, dst,
                  ew, feat_split=True)
    s2 = _matmul_relu(a1, b1.reshape(2, 128), W2)
    a2 = _spmm_sc(s2.reshape(2 * N_NODES, 128), src2, dst,
                  ew, feat_split=True)
    s3 = _matmul_relu(a2, b2.reshape(2, 128), W3)
    p3 = _spmm_sc(s3, src, dst, ew, feat_split=False)
    return _combine_bias(p3, b3.reshape(1, 128))


# speedup vs baseline: 2.9737x; 1.5419x over previous
import functools

import jax
import jax.numpy as jnp
from jax import lax
from jax.experimental import pallas as pl
from jax.experimental.pallas import tpu as pltpu
from jax.experimental.pallas import tpu_sc as plsc

N_NODES = 10000
N_EDGES = 160000
EPAD = 163840
NC = 2
NS = 16
LANES = 16
INIT_ROWS = 125


def _matmul_x(x, w, bm=1000):
    n, k = x.shape
    d = w.shape[1]

    def body(x_ref, w_ref, o_ref):
        o_ref[...] = jnp.dot(x_ref[...], w_ref[...],
                             preferred_element_type=jnp.float32)

    return pl.pallas_call(
        body,
        grid=(n // bm,),
        in_specs=[
            pl.BlockSpec((bm, k), lambda i: (i, 0)),
            pl.BlockSpec((k, d), lambda i: (0, 0)),
        ],
        out_specs=pl.BlockSpec((bm, d), lambda i: (i, 0)),
        out_shape=jax.ShapeDtypeStruct((n, d), jnp.float32),
    )(x, w)


def _matmul_relu(a2, b2, w, bm=1000):
    _, n, h = a2.shape
    d = w.shape[1]

    def body(a_ref, b_ref, w_ref, o_ref):
        h0 = jnp.maximum(a_ref[0] + b_ref[0], 0.0)
        h1 = jnp.maximum(a_ref[1] + b_ref[1], 0.0)
        o_ref[...] = (
            jnp.dot(h0, w_ref[:h, :], preferred_element_type=jnp.float32)
            + jnp.dot(h1, w_ref[h:, :], preferred_element_type=jnp.float32)
        )

    return pl.pallas_call(
        body,
        grid=(n // bm,),
        in_specs=[
            pl.BlockSpec((2, bm, h), lambda i: (0, i, 0)),
            pl.BlockSpec((2, h), lambda i: (0, 0)),
            pl.BlockSpec((2 * h, d), lambda i: (0, 0)),
        ],
        out_specs=pl.BlockSpec((bm, d), lambda i: (i, 0)),
        out_shape=jax.ShapeDtypeStruct((n, d), jnp.float32),
    )(a2, b2, w)


def _combine_bias(p2, b, bm=1000):
    _, n, d = p2.shape

    def body(p_ref, b_ref, o_ref):
        o_ref[...] = p_ref[0] + p_ref[1] + b_ref[...]

    return pl.pallas_call(
        body,
        grid=(n // bm,),
        in_specs=[
            pl.BlockSpec((2, bm, d), lambda i: (0, i, 0)),
            pl.BlockSpec((1, d), lambda i: (0, 0)),
        ],
        out_specs=pl.BlockSpec((bm, d), lambda i: (i, 0)),
        out_shape=jax.ShapeDtypeStruct((n, d), jnp.float32),
    )(p2, b)


def _spmm_sc(sup, src2d, dst2d, ew2d, feat_split):
    dh = sup.shape[1]
    dblks = dh // LANES
    chunk = 128
    nbuf = 2
    if feat_split:
        per_tile = EPAD // NS
    else:
        per_tile = EPAD // (NC * NS)
    nchunks = per_tile // chunk
    rows_per_tile = N_NODES // NS
    mesh = plsc.VectorSubcoreMesh(core_axis_name="c", subcore_axis_name="s",
                                  num_cores=NC, num_subcores=NS)

    @functools.partial(
        pl.kernel,
        out_type=jax.ShapeDtypeStruct((NC, N_NODES, dh), jnp.float32),
        mesh=mesh,
        scratch_types=(
            [pltpu.VMEM((chunk, dh), jnp.float32) for _ in range(nbuf)]
            + [pltpu.VMEM((chunk,), jnp.int32) for _ in range(nbuf)]
            + [pltpu.VMEM((chunk,), jnp.float32) for _ in range(nbuf)]
            + [
                pltpu.VMEM((nchunks, chunk), jnp.int32),
                pltpu.VMEM_SHARED((N_NODES, dh), jnp.float32),
            ]
            + [pltpu.SemaphoreType.DMA for _ in range(3 * nbuf)]
        ),
    )
    def k(sup_hbm, src_hbm, dst_hbm, ew_hbm, out_hbm, *scr):
        rows = scr[:nbuf]
        si = scr[nbuf:2 * nbuf]
        wv = scr[2 * nbuf:3 * nbuf]
        di_v, acc_sh = scr[3 * nbuf:3 * nbuf + 2]
        gsem = scr[3 * nbuf + 2:3 * nbuf + 2 + nbuf]
        ssem = scr[3 * nbuf + 2 + nbuf:3 * nbuf + 2 + 2 * nbuf]
        isem = scr[3 * nbuf + 2 + 2 * nbuf:]
        c = lax.axis_index("c")
        s = lax.axis_index("s")

        def zero_row(r, carry):
            for d in range(dblks):
                rows[0][r, pl.ds(d * LANES, LANES)] = jnp.zeros(
                    (LANES,), jnp.float32)
            return carry

        lax.fori_loop(0, chunk, zero_row, 0)
        row0 = s * rows_per_tile
        for j in range(rows_per_tile // chunk):
            pltpu.sync_copy(rows[0], acc_sh.at[pl.ds(row0 + j * chunk,
                                                     chunk)])
        rem = rows_per_tile % chunk
        pltpu.sync_copy(
            rows[0].at[pl.ds(0, rem)],
            acc_sh.at[pl.ds(row0 + rows_per_tile - rem, rem)])

        if feat_split:
            erow0 = s * nchunks
            srow0 = c * (EPAD // chunk) + erow0
        else:
            erow0 = (c * NS + s) * nchunks
            srow0 = erow0
        pltpu.sync_copy(dst_hbm.at[pl.ds(erow0, nchunks)], di_v)
        plsc.subcore_barrier()

        def stage_idx(b, kk):
            pltpu.async_copy(src_hbm.at[srow0 + kk], si[b], isem[b])
            pltpu.async_copy(ew_hbm.at[erow0 + kk], wv[b], isem[b])

        def wait_idx(b):
            pltpu.make_async_copy(src_hbm.at[0], si[b], isem[b]).wait()
            pltpu.make_async_copy(ew_hbm.at[0], wv[b], isem[b]).wait()

        def scale(b):
            def scale_group(g, cc):
                wvec = wv[b][pl.ds(g * LANES, LANES)]
                for j in range(LANES):
                    e = g * LANES + j
                    wj = wvec[j]
                    for d in range(dblks):
                        sl = pl.ds(d * LANES, LANES)
                        rows[b][e, sl] = rows[b][e, sl] * wj
                return cc

            lax.fori_loop(0, chunk // LANES, scale_group, 0)

        for b in range(nbuf):
            stage_idx(b, b)
            wait_idx(b)
            pltpu.async_copy(sup_hbm.at[si[b]], rows[b], gsem[b])

        @pl.loop(0, nchunks // nbuf)
        def _round(r):
            for b in range(nbuf):
                pltpu.make_async_copy(sup_hbm.at[si[b]], rows[b],
                                      gsem[b]).wait()
                scale(b)
                kk = r * nbuf + b
                pltpu.async_copy(rows[b], acc_sh.at[di_v.at[kk]], ssem[b],
                                 add=True)

            for b in range(nbuf):
                kk = (r + 1) * nbuf + b
                kw = jnp.where(kk < nchunks, kk, 0)
                stage_idx(b, kw)
                pltpu.make_async_copy(rows[b], acc_sh.at[di_v.at[0]],
                                      ssem[b]).wait()
                wait_idx(b)
                pltpu.async_copy(sup_hbm.at[si[b]], rows[b], gsem[b])

        for b in range(nbuf):
            pltpu.make_async_copy(sup_hbm.at[si[b]], rows[b],
                                  gsem[b]).wait()
        plsc.subcore_barrier()

        out_rows = rows_per_tile // 8 * 8
        pltpu.sync_copy(acc_sh.at[pl.ds(s * out_rows, out_rows)],
                        out_hbm.at[c, pl.ds(s * out_rows, out_rows)])
        tail = N_NODES - NS * out_rows

        @pl.when(s == 0)
        def _():
            pltpu.sync_copy(acc_sh.at[pl.ds(NS * out_rows, tail)],
                            out_hbm.at[c, pl.ds(NS * out_rows, tail)])

    return k(sup, src2d, dst2d, ew2d)


def kernel(x, edge_index, edge_weight, W1, b1, W2, b2, W3, b3):
    ei = edge_index.astype(jnp.int32)
    pad = EPAD - N_EDGES
    src = jnp.concatenate([ei[0], jnp.zeros((pad,), jnp.int32)])
    dst = jnp.concatenate([ei[1], jnp.zeros((pad,), jnp.int32)])
    ew = jnp.concatenate([edge_weight, jnp.zeros((pad,), jnp.float32)])
    src2 = jnp.concatenate([src * 2, src * 2 + 1]).reshape(-1, 128)
    src1 = src.reshape(-1, 128)
    dst2d = dst.reshape(-1, 128)
    ew2d = ew.reshape(-1, 128)

    s1 = _matmul_x(x, W1)
    a1 = _spmm_sc(s1.reshape(2 * N_NODES, 128), src2, dst2d,
                  ew2d, feat_split=True)
    s2 = _matmul_relu(a1, b1.reshape(2, 128), W2)
    a2 = _spmm_sc(s2.reshape(2 * N_NODES, 128), src2, dst2d,
                  ew2d, feat_split=True)
    s3 = _matmul_relu(a2, b2.reshape(2, 128), W3)
    p3 = _spmm_sc(s3, src1, dst2d, ew2d, feat_split=False)
    return _combine_bias(p3, b3.reshape(1, 128))
